# baseline (speedup 1.0000x reference)
import functools

import jax
import jax.numpy as jnp
from jax import lax
from jax.experimental import pallas as pl
from jax.experimental.pallas import tpu as pltpu
from jax.experimental.pallas import tpu_sc as plsc

N_PAD = 10112
D = 128
NREL = 8
NC, NS, L = 2, 16, 16
CHUNK = 128
ROWS_PER_TILE = N_PAD // NS


def _matmul_body(x_ref, w_ref, y_ref):
    y_ref[0] = jnp.dot(x_ref[...], w_ref[0], preferred_element_type=jnp.float32)


MM_BLK = N_PAD // 8


def _matmul(x_pad, w_pair):
    return pl.pallas_call(
        _matmul_body,
        grid=(8, w_pair.shape[0]),
        in_specs=[
            pl.BlockSpec((MM_BLK, D), lambda i, r: (i, 0)),
            pl.BlockSpec((1, D, 2 * D), lambda i, r: (r, 0, 0)),
        ],
        out_specs=pl.BlockSpec((1, MM_BLK, 2 * D), lambda i, r: (r, i, 0)),
        out_shape=jax.ShapeDtypeStruct((w_pair.shape[0], N_PAD, 2 * D),
                                       jnp.float32),
    )(x_pad, w_pair)


NBUF = 2
C0 = 64
C1 = 16
ALLOC_CHUNKS = NS * C0 + (NS - 1) * C1 + C0


def _sc_body(table_hbm, zero_hbm, gidx_hbm, dst_hbm, out_hbm,
             gidx_v, dst_v, rows0, rows1, acc_sh, sem0, sem1):
    rows_b = (rows0, rows1)
    sem_b = (sem0, sem1)
    cid = lax.axis_index("c")
    sid = lax.axis_index("s")
    rows_sl = pl.ds(sid * ROWS_PER_TILE, ROWS_PER_TILE)

    pltpu.sync_copy(zero_hbm.at[rows_sl], acc_sh.at[rows_sl])
    plsc.subcore_barrier()

    nch = jnp.where(cid == 0, C0, C1)
    base = pl.multiple_of(jnp.where(cid == 0, sid * C0, NS * C0 + sid * C1), 8)

    pltpu.sync_copy(gidx_hbm.at[pl.ds(base, C0)], gidx_v)
    pltpu.sync_copy(dst_hbm.at[pl.ds(base, C0)], dst_v)

    def _grp(g, _):
        descs = []
        for b in range(NBUF):
            c = NBUF * g + b
            descs.append(pltpu.async_copy(
                table_hbm.at[gidx_v.at[c]], rows_b[b], sem_b[b]))
        for b in range(NBUF):
            c = NBUF * g + b
            descs[b].wait()
            pltpu.sync_copy(rows_b[b], acc_sh.at[dst_v.at[c]], add=True)
        return 0
    lax.fori_loop(0, nch // NBUF, _grp, 0)

    plsc.subcore_barrier()
    pltpu.sync_copy(acc_sh.at[rows_sl], out_hbm.at[cid, rows_sl])


_sc_scatter = functools.partial(
    pl.kernel,
    out_type=jax.ShapeDtypeStruct((NC, N_PAD, D), jnp.float32),
    mesh=plsc.VectorSubcoreMesh(core_axis_name="c", subcore_axis_name="s",
                                num_cores=NC, num_subcores=NS),
    scratch_types=[
        pltpu.VMEM((C0, CHUNK), jnp.int32),
        pltpu.VMEM((C0, CHUNK), jnp.int32),
        pltpu.VMEM((CHUNK, D), jnp.float32),
        pltpu.VMEM((CHUNK, D), jnp.float32),
        pltpu.VMEM_SHARED((N_PAD, D), jnp.float32),
        pltpu.SemaphoreType.DMA,
        pltpu.SemaphoreType.DMA,
    ],
)(_sc_body)


def _combine_body(p_ref, s_ref, o_ref):
    o_ref[...] = jnp.maximum(p_ref[0] + p_ref[1] + s_ref[0, :, :D], 0.0)


def _combine(partials, y_pair, n):
    blk = n // 5
    return pl.pallas_call(
        _combine_body,
        grid=(5,),
        in_specs=[
            pl.BlockSpec((NC, blk, D), lambda i: (0, i, 0)),
            pl.BlockSpec((1, blk, 2 * D), lambda i: (NREL // 2, i, 0)),
        ],
        out_specs=pl.BlockSpec((blk, D), lambda i: (i, 0)),
        out_shape=jax.ShapeDtypeStruct((n, D), jnp.float32),
    )(partials, y_pair)


def kernel(x, weight, self_loop_w, edge_index, edge_type):
    n = x.shape[0]
    ne = edge_type.shape[0]
    x_pad = jnp.pad(x, ((0, N_PAD - n), (0, 0)))
    w_all = jnp.concatenate(
        [weight, self_loop_w.T[None], jnp.zeros((1, D, D), x.dtype)], axis=0)
    w_pair = (w_all.reshape(5, 2, D, D).transpose(0, 2, 1, 3)
              .reshape(5, D, 2 * D))
    y_pair = _matmul(x_pad, w_pair)
    table = y_pair.reshape(5 * N_PAD * 2, D)

    pad = ALLOC_CHUNKS * CHUNK - ne
    gidx = ((edge_type // 2) * (2 * N_PAD) + 2 * edge_index[0]
            + (edge_type % 2))
    gidx_p = jnp.pad(gidx, (0, pad)).reshape(ALLOC_CHUNKS, CHUNK)
    dst_p = jnp.pad(edge_index[1], (0, pad),
                    constant_values=n).reshape(ALLOC_CHUNKS, CHUNK)

    zero = jnp.zeros((N_PAD, D), jnp.float32)
    partials = _sc_scatter(table, zero, gidx_p, dst_p)
    return _combine(partials, y_pair, n)


# speedup vs baseline: 1.2335x; 1.2335x over previous
import functools

import jax
import jax.numpy as jnp
from jax import lax
from jax.experimental import pallas as pl
from jax.experimental.pallas import tpu as pltpu
from jax.experimental.pallas import tpu_sc as plsc

N_PAD = 10112
D = 128
NREL = 8
NC, NS, L = 2, 16, 16
CHUNK = 128
ROWS_PER_TILE = N_PAD // NS


def _matmul_body(x_ref, w_ref, y_ref):
    y_ref[0] = jnp.dot(x_ref[...], w_ref[0], preferred_element_type=jnp.float32)


MM_BLK = N_PAD // 8


def _matmul(x_pad, w_all):
    return pl.pallas_call(
        _matmul_body,
        grid=(8, w_all.shape[0]),
        in_specs=[
            pl.BlockSpec((MM_BLK, D), lambda i, r: (i, 0)),
            pl.BlockSpec((1, D, D), lambda i, r: (r, 0, 0)),
        ],
        out_specs=pl.BlockSpec((1, MM_BLK, D), lambda i, r: (r, i, 0)),
        out_shape=jax.ShapeDtypeStruct((w_all.shape[0], N_PAD, D), jnp.float32),
    )(x_pad, w_all)


NBUF = 2
C0 = 64
C1 = 16
ALLOC_CHUNKS = NS * C0 + (NS - 1) * C1 + C0


def _sc_body(table_hbm, zero_hbm, gidx_hbm, dst_hbm, out_hbm,
             gidx_v, dst_v, rows0, rows1, acc_sh, sem0, sem1):
    rows_b = (rows0, rows1)
    sem_b = (sem0, sem1)
    cid = lax.axis_index("c")
    sid = lax.axis_index("s")
    rows_sl = pl.ds(sid * ROWS_PER_TILE, ROWS_PER_TILE)

    @pl.when(cid == 0)
    def _():
        init_base = NREL * N_PAD + sid * ROWS_PER_TILE
        pltpu.sync_copy(table_hbm.at[pl.ds(init_base, ROWS_PER_TILE)],
                        acc_sh.at[rows_sl])

    @pl.when(cid == 1)
    def _():
        pltpu.sync_copy(zero_hbm.at[rows_sl], acc_sh.at[rows_sl])

    plsc.subcore_barrier()

    nch = jnp.where(cid == 0, C0, C1)
    base = pl.multiple_of(jnp.where(cid == 0, sid * C0, NS * C0 + sid * C1), 8)

    pltpu.sync_copy(gidx_hbm.at[pl.ds(base, C0)], gidx_v)
    pltpu.sync_copy(dst_hbm.at[pl.ds(base, C0)], dst_v)

    def _grp(g, _):
        descs = []
        for b in range(NBUF):
            c = NBUF * g + b
            descs.append(pltpu.async_copy(
                table_hbm.at[gidx_v.at[c]], rows_b[b], sem_b[b]))
        for b in range(NBUF):
            c = NBUF * g + b
            descs[b].wait()
            pltpu.sync_copy(rows_b[b], acc_sh.at[dst_v.at[c]], add=True)
        return 0
    lax.fori_loop(0, nch // NBUF, _grp, 0)

    plsc.subcore_barrier()
    pltpu.sync_copy(acc_sh.at[rows_sl], out_hbm.at[cid, rows_sl])


_sc_scatter = functools.partial(
    pl.kernel,
    out_type=jax.ShapeDtypeStruct((NC, N_PAD, D), jnp.float32),
    mesh=plsc.VectorSubcoreMesh(core_axis_name="c", subcore_axis_name="s",
                                num_cores=NC, num_subcores=NS),
    scratch_types=[
        pltpu.VMEM((C0, CHUNK), jnp.int32),
        pltpu.VMEM((C0, CHUNK), jnp.int32),
        pltpu.VMEM((CHUNK, D), jnp.float32),
        pltpu.VMEM((CHUNK, D), jnp.float32),
        pltpu.VMEM_SHARED((N_PAD, D), jnp.float32),
        pltpu.SemaphoreType.DMA,
        pltpu.SemaphoreType.DMA,
    ],
)(_sc_body)


def _combine_body(p_ref, o_ref):
    o_ref[...] = jnp.maximum(p_ref[0] + p_ref[1], 0.0)


def _combine(partials, n):
    blk = n // 5
    return pl.pallas_call(
        _combine_body,
        grid=(5,),
        in_specs=[pl.BlockSpec((NC, blk, D), lambda i: (0, i, 0))],
        out_specs=pl.BlockSpec((blk, D), lambda i: (i, 0)),
        out_shape=jax.ShapeDtypeStruct((n, D), jnp.float32),
    )(partials)


def kernel(x, weight, self_loop_w, edge_index, edge_type):
    n = x.shape[0]
    ne = edge_type.shape[0]
    x_pad = jnp.pad(x, ((0, N_PAD - n), (0, 0)))
    w_all = jnp.concatenate([weight, self_loop_w.T[None]], axis=0)
    table = _matmul(x_pad, w_all).reshape(w_all.shape[0] * N_PAD, D)

    pad = ALLOC_CHUNKS * CHUNK - ne
    gidx = edge_type * N_PAD + edge_index[0]
    gidx_p = jnp.pad(gidx, (0, pad)).reshape(ALLOC_CHUNKS, CHUNK)
    dst_p = jnp.pad(edge_index[1], (0, pad),
                    constant_values=n).reshape(ALLOC_CHUNKS, CHUNK)

    zero = jnp.zeros((N_PAD, D), jnp.float32)
    partials = _sc_scatter(table, zero, gidx_p, dst_p)
    return _combine(partials, n)
